# baseline (speedup 1.0000x reference)
import functools

import jax
import jax.numpy as jnp
from jax import lax
from jax.experimental import pallas as pl
from jax.experimental.pallas import tpu as pltpu
from jax.experimental.pallas import tpu_sc as plsc

D = 32
L = 16
NC, NS = 2, 16
NW = NC * NS
CHUNK = 128


@functools.partial(jax.jit, static_argnames=("b",))
def _mf_sc(user2d, item2d, ulat, ilat, b):
    b_per_w = b // NW
    n_chunks = b_per_w // CHUNK
    n_groups = b_per_w // L

    mesh = plsc.VectorSubcoreMesh(core_axis_name="c", subcore_axis_name="s")

    @functools.partial(
        pl.kernel,
        out_type=(
            jax.ShapeDtypeStruct((b, D), jnp.float32),
            jax.ShapeDtypeStruct((b, D), jnp.float32),
            jax.ShapeDtypeStruct((b,), jnp.float32),
        ),
        mesh=mesh,
        compiler_params=pltpu.CompilerParams(
            needs_layout_passes=False, use_tc_tiling_on_sc=False),
        scratch_types=[
            pltpu.VMEM((n_chunks, CHUNK), jnp.int32),
            pltpu.VMEM((n_chunks, CHUNK), jnp.int32),
            pltpu.VMEM((b_per_w, D), jnp.float32),
            pltpu.VMEM((b_per_w, D), jnp.float32),
            pltpu.VMEM((b_per_w,), jnp.float32),
            pltpu.SemaphoreType.DMA,
            pltpu.SemaphoreType.DMA,
        ],
    )
    def k(user_h, item_h, ulat_h, ilat_h, out_u, out_i, out_r,
          idx_u, idx_i, u_rows, i_rows, res_v, gsem, wsem):
        wid = lax.axis_index("s") * NC + lax.axis_index("c")
        base = wid * b_per_w
        crow = wid * n_chunks

        pltpu.sync_copy(user_h.at[pl.ds(crow, n_chunks)], idx_u)
        pltpu.sync_copy(item_h.at[pl.ds(crow, n_chunks)], idx_i)

        copies = []
        for j in range(n_chunks):
            copies.append(pltpu.async_copy(
                ulat_h.at[idx_u.at[j]], u_rows.at[pl.ds(j * CHUNK, CHUNK)], gsem))
            copies.append(pltpu.async_copy(
                ilat_h.at[idx_i.at[j]], i_rows.at[pl.ds(j * CHUNK, CHUNK)], gsem))
        for c in copies:
            c.wait()

        wb_u = pltpu.async_copy(u_rows, out_u.at[pl.ds(base, b_per_w)], wsem)
        wb_i = pltpu.async_copy(i_rows, out_i.at[pl.ds(base, b_per_w)], wsem)

        lane = lax.iota(jnp.int32, L)

        @pl.loop(0, n_groups)
        def _(g):
            r0 = g * L
            acc = jnp.zeros((L,), jnp.float32)
            for k in range(L):
                r = r0 + k
                t = (u_rows[r, pl.ds(0, L)] * i_rows[r, pl.ds(0, L)]
                     + u_rows[r, pl.ds(L, L)] * i_rows[r, pl.ds(L, L)])
                acc = jnp.where(lane == k, jnp.sum(t), acc)
            res_v[pl.ds(r0, L)] = acc

        pltpu.sync_copy(res_v, out_r.at[pl.ds(base, b_per_w)])
        wb_u.wait()
        wb_i.wait()

    return k(user2d, item2d, ulat, ilat)


def kernel(user, item, his_r, rct_r, user_bias_w, item_bias_w,
           user_laten_w, item_laten_w):
    b = user.shape[0]
    user2d = user.reshape(b // CHUNK, CHUNK)
    item2d = item.reshape(b // CHUNK, CHUNK)
    return _mf_sc(user2d, item2d, user_laten_w, item_laten_w, b)


# speedup vs baseline: 1.4806x; 1.4806x over previous
import functools

import jax
import jax.numpy as jnp
from jax import lax
from jax.experimental import pallas as pl
from jax.experimental.pallas import tpu as pltpu
from jax.experimental.pallas import tpu_sc as plsc

D = 32
L = 16
NC, NS = 2, 16
NW = NC * NS
K = 16
H = 2


@functools.partial(jax.jit, static_argnames=("b",))
def _mf_sc(user, item, ulat, ilat, b):
    b_per_w = b // NW
    rows_h = b_per_w // H
    nblk = rows_h // K

    mesh = plsc.VectorSubcoreMesh(core_axis_name="c", subcore_axis_name="s")

    @functools.partial(
        pl.kernel,
        out_type=(
            jax.ShapeDtypeStruct((b, D), jnp.float32),
            jax.ShapeDtypeStruct((b, D), jnp.float32),
            jax.ShapeDtypeStruct((b,), jnp.float32),
        ),
        mesh=mesh,
        compiler_params=pltpu.CompilerParams(needs_layout_passes=False),
        scratch_types=[
            pltpu.VMEM((b // NW,), jnp.int32),
            pltpu.VMEM((b // NW,), jnp.int32),
            pltpu.VMEM((b // NW // H, D), jnp.float32),
            pltpu.VMEM((b // NW // H, D), jnp.float32),
            pltpu.VMEM((b // NW,), jnp.float32),
            pltpu.SemaphoreType.DMA,
            pltpu.SemaphoreType.DMA,
        ],
    )
    def k(user_h, item_h, ulat_h, ilat_h, out_u, out_i, out_r,
          idx_us, idx_is, u_rows, i_rows, res_v, gsem, wsem):
        wid = lax.axis_index("s") * NC + lax.axis_index("c")
        base = wid * b_per_w

        pltpu.sync_copy(user_h.at[pl.ds(base, b_per_w)], idx_us)
        pltpu.sync_copy(item_h.at[pl.ds(base, b_per_w)], idx_is)

        lane = lax.iota(jnp.int32, L)

        for h in range(H):
            hbase = h * rows_h

            def fire(blk, hbase=hbase):
                rr0 = blk * K
                uvec = idx_us[pl.ds(hbase + rr0, K)]
                ivec = idx_is[pl.ds(hbase + rr0, K)]
                for kk in range(K):
                    rr = rr0 + kk
                    pltpu.async_copy(ulat_h.at[uvec[kk]], u_rows.at[rr], gsem)
                    pltpu.async_copy(ilat_h.at[ivec[kk]], i_rows.at[rr], gsem)

            def drain_batch():
                pltpu.make_async_copy(
                    ulat_h.at[pl.ds(0, 2 * K)], u_rows.at[pl.ds(0, 2 * K)],
                    gsem).wait()

            fire(0)

            @pl.loop(1, nblk)
            def _(blk):
                fire(blk)
                drain_batch()

            drain_batch()

            wb_u = pltpu.async_copy(
                u_rows, out_u.at[pl.ds(base + hbase, rows_h)], wsem)
            wb_i = pltpu.async_copy(
                i_rows, out_i.at[pl.ds(base + hbase, rows_h)], wsem)

            @pl.loop(0, rows_h // L)
            def _(g):
                r0 = g * L
                acc = jnp.zeros((L,), jnp.float32)
                for kk in range(L):
                    rr = r0 + kk
                    t = (u_rows[rr, pl.ds(0, L)] * i_rows[rr, pl.ds(0, L)]
                         + u_rows[rr, pl.ds(L, L)] * i_rows[rr, pl.ds(L, L)])
                    acc = jnp.where(lane == kk, jnp.sum(t), acc)
                res_v[pl.ds(hbase + r0, L)] = acc

            wb_u.wait()
            wb_i.wait()

        pltpu.sync_copy(res_v, out_r.at[pl.ds(base, b_per_w)])

    return k(user, item, ulat, ilat)


def kernel(user, item, his_r, rct_r, user_bias_w, item_bias_w,
           user_laten_w, item_laten_w):
    return _mf_sc(user, item, user_laten_w, item_laten_w, user.shape[0])
